# baseline (speedup 1.0000x reference)
import functools

import jax
import jax.numpy as jnp
from jax.experimental import pallas as pl
from jax.experimental.pallas import tpu as pltpu
from jax.experimental.pallas import tpu_sc as plsc

HIDDEN = 128
EPS = 1e-12
GATHER_WINDOW = 128
BB = 32


def _sc_gather(table, flat_ids):
    n = flat_ids.shape[0]
    idx = flat_ids.reshape(1, n)
    mesh = plsc.VectorSubcoreMesh(core_axis_name="c", subcore_axis_name="s")

    @functools.partial(
        pl.kernel,
        out_type=jax.ShapeDtypeStruct((n, HIDDEN), table.dtype),
        mesh=mesh,
    )
    def gather_kernel(table_hbm, idx_hbm, out_hbm):
        def body(idx_vmem, out_vmem):
            pltpu.sync_copy(table_hbm.at[idx_vmem.at[0]], out_vmem)

        pltpu.emit_pipeline(
            body,
            grid=(n // GATHER_WINDOW,),
            in_specs=[pl.BlockSpec((1, GATHER_WINDOW), index_map=lambda i: (0, i))],
            out_specs=[pl.BlockSpec((GATHER_WINDOW, HIDDEN),
                                    index_map=lambda i: (i, 0))],
            core_axis_name=("c", "s"),
            dimension_semantics=(pltpu.PARALLEL,),
        )(idx_hbm, out_hbm)

    return gather_kernel(table, idx)


def _addln_body(g_ref, tt_ref, base_ref, diff_ref, w_ref, b_ref, o_ref):
    x = g_ref[...]
    x = x + base_ref[...] + tt_ref[...] * diff_ref[...]
    mean = jnp.mean(x, axis=-1, keepdims=True)
    xc = x - mean
    var = jnp.mean(xc * xc, axis=-1, keepdims=True)
    inv = jax.lax.rsqrt(var + EPS)
    o_ref[...] = xc * inv * w_ref[...] + b_ref[...]


def _tc_addln(gathered, tt_f, base, diff, ln_w, ln_b, interpret=False):
    b, s, h = gathered.shape
    return pl.pallas_call(
        _addln_body,
        grid=(b // BB,),
        in_specs=[
            pl.BlockSpec((BB, s, h), lambda i: (i, 0, 0)),
            pl.BlockSpec((BB, s, 1), lambda i: (i, 0, 0)),
            pl.BlockSpec((1, s, h), lambda i: (0, 0, 0)),
            pl.BlockSpec((1, 1, h), lambda i: (0, 0, 0)),
            pl.BlockSpec((1, 1, h), lambda i: (0, 0, 0)),
            pl.BlockSpec((1, 1, h), lambda i: (0, 0, 0)),
        ],
        out_specs=pl.BlockSpec((BB, s, h), lambda i: (i, 0, 0)),
        out_shape=jax.ShapeDtypeStruct((b, s, h), jnp.float32),
        interpret=interpret,
    )(gathered, tt_f, base, diff, ln_w, ln_b)


NCHUNKS = 4


def kernel(input_ids, token_type_ids, W_word, W_pos, W_type, ln_w, ln_b):
    b, s = input_ids.shape
    ids = input_ids.astype(jnp.int32)
    tt_f = token_type_ids.astype(jnp.float32)[:, :, None]
    base = (W_pos[:s] + W_type[0][None, :])[None, :, :]
    diff = (W_type[1] - W_type[0])[None, None, :]
    lnw = ln_w.reshape(1, 1, HIDDEN)
    lnb = ln_b.reshape(1, 1, HIDDEN)
    cb = b // NCHUNKS
    outs = []
    for c in range(NCHUNKS):
        flat_c = ids[c * cb:(c + 1) * cb].reshape(-1)
        g = _sc_gather(W_word, flat_c).reshape(cb, s, HIDDEN)
        outs.append(_tc_addln(g, tt_f[c * cb:(c + 1) * cb], base, diff, lnw, lnb))
    return jnp.concatenate(outs, axis=0)


# speedup vs baseline: 1.3723x; 1.3723x over previous
import functools

import jax
import jax.numpy as jnp
from jax.experimental import pallas as pl
from jax.experimental.pallas import tpu as pltpu
from jax.experimental.pallas import tpu_sc as plsc

HIDDEN = 128
EPS = 1e-12
GATHER_WINDOW = 128
BB = 32


KWIN = 2


def _sc_gather(table, flat_ids):
    n = flat_ids.shape[0]
    idx = flat_ids.reshape(1, n // GATHER_WINDOW, GATHER_WINDOW)
    mesh = plsc.VectorSubcoreMesh(core_axis_name="c", subcore_axis_name="s")

    @functools.partial(
        pl.kernel,
        out_type=jax.ShapeDtypeStruct((n, HIDDEN), table.dtype),
        mesh=mesh,
        scratch_types=[pltpu.SemaphoreType.DMA] * KWIN,
    )
    def gather_kernel(table_hbm, idx_hbm, out_hbm, *sems):
        def body(idx_vmem, out_vmem):
            copies = []
            for j in range(KWIN):
                copies.append(pltpu.async_copy(
                    table_hbm.at[idx_vmem.at[0, j]],
                    out_vmem.at[pl.ds(j * GATHER_WINDOW, GATHER_WINDOW)],
                    sems[j]))
            for c in copies:
                c.wait()

        pltpu.emit_pipeline(
            body,
            grid=(n // (KWIN * GATHER_WINDOW),),
            in_specs=[pl.BlockSpec((1, KWIN, GATHER_WINDOW),
                                   index_map=lambda i: (0, i, 0))],
            out_specs=[pl.BlockSpec((KWIN * GATHER_WINDOW, HIDDEN),
                                    index_map=lambda i: (i, 0))],
            core_axis_name=("c", "s"),
            dimension_semantics=(pltpu.PARALLEL,),
        )(idx_hbm, out_hbm)

    return gather_kernel(table, idx)


def _addln_body(g_ref, tt_ref, base_ref, diff_ref, w_ref, b_ref, o_ref):
    x = g_ref[...]
    x = x + base_ref[...] + tt_ref[...] * diff_ref[...]
    mean = jnp.mean(x, axis=-1, keepdims=True)
    xc = x - mean
    var = jnp.mean(xc * xc, axis=-1, keepdims=True)
    inv = jax.lax.rsqrt(var + EPS)
    o_ref[...] = xc * inv * w_ref[...] + b_ref[...]


def _tc_addln(gathered, tt_f, base, diff, ln_w, ln_b, interpret=False):
    b, s, h = gathered.shape
    return pl.pallas_call(
        _addln_body,
        grid=(b // BB,),
        in_specs=[
            pl.BlockSpec((BB, s, h), lambda i: (i, 0, 0)),
            pl.BlockSpec((BB, s, 1), lambda i: (i, 0, 0)),
            pl.BlockSpec((1, s, h), lambda i: (0, 0, 0)),
            pl.BlockSpec((1, 1, h), lambda i: (0, 0, 0)),
            pl.BlockSpec((1, 1, h), lambda i: (0, 0, 0)),
            pl.BlockSpec((1, 1, h), lambda i: (0, 0, 0)),
        ],
        out_specs=pl.BlockSpec((BB, s, h), lambda i: (i, 0, 0)),
        out_shape=jax.ShapeDtypeStruct((b, s, h), jnp.float32),
        interpret=interpret,
    )(gathered, tt_f, base, diff, ln_w, ln_b)


def kernel(input_ids, token_type_ids, W_word, W_pos, W_type, ln_w, ln_b):
    b, s = input_ids.shape
    flat_ids = input_ids.reshape(-1).astype(jnp.int32)
    gathered = _sc_gather(W_word, flat_ids).reshape(b, s, HIDDEN)
    tt_f = token_type_ids.astype(jnp.float32)[:, :, None]
    base = (W_pos[:s] + W_type[0][None, :])[None, :, :]
    diff = (W_type[1] - W_type[0])[None, None, :]
    return _tc_addln(gathered, tt_f, base, diff,
                     ln_w.reshape(1, 1, HIDDEN), ln_b.reshape(1, 1, HIDDEN))


# speedup vs baseline: 1.4290x; 1.0413x over previous
import functools

import jax
import jax.numpy as jnp
from jax.experimental import pallas as pl
from jax.experimental.pallas import tpu as pltpu
from jax.experimental.pallas import tpu_sc as plsc

HIDDEN = 128
EPS = 1e-12
GATHER_WINDOW = 128
BB = 64


KWIN = 2


def _sc_gather(table, flat_ids):
    n = flat_ids.shape[0]
    idx = flat_ids.reshape(1, n // GATHER_WINDOW, GATHER_WINDOW)
    mesh = plsc.VectorSubcoreMesh(core_axis_name="c", subcore_axis_name="s")

    @functools.partial(
        pl.kernel,
        out_type=jax.ShapeDtypeStruct((n, HIDDEN), table.dtype),
        mesh=mesh,
        scratch_types=[pltpu.SemaphoreType.DMA] * KWIN,
    )
    def gather_kernel(table_hbm, idx_hbm, out_hbm, *sems):
        def body(idx_vmem, out_vmem):
            copies = []
            for j in range(KWIN):
                copies.append(pltpu.async_copy(
                    table_hbm.at[idx_vmem.at[0, j]],
                    out_vmem.at[pl.ds(j * GATHER_WINDOW, GATHER_WINDOW)],
                    sems[j]))
            for c in copies:
                c.wait()

        pltpu.emit_pipeline(
            body,
            grid=(n // (KWIN * GATHER_WINDOW),),
            in_specs=[pl.BlockSpec((1, KWIN, GATHER_WINDOW),
                                   index_map=lambda i: (0, i, 0))],
            out_specs=[pl.BlockSpec((KWIN * GATHER_WINDOW, HIDDEN),
                                    index_map=lambda i: (i, 0))],
            core_axis_name=("c", "s"),
            dimension_semantics=(pltpu.PARALLEL,),
        )(idx_hbm, out_hbm)

    return gather_kernel(table, idx)


def _addln_body(g_ref, tt_ref, base_ref, diff_ref, w_ref, b_ref, o_ref):
    x = g_ref[...]
    x = x + base_ref[...] + tt_ref[...] * diff_ref[...]
    mean = jnp.mean(x, axis=-1, keepdims=True)
    xc = x - mean
    var = jnp.mean(xc * xc, axis=-1, keepdims=True)
    inv = jax.lax.rsqrt(var + EPS)
    o_ref[...] = xc * inv * w_ref[...] + b_ref[...]


def _tc_addln(gathered, tt_f, base, diff, ln_w, ln_b, interpret=False):
    b, s, h = gathered.shape
    return pl.pallas_call(
        _addln_body,
        grid=(b // BB,),
        in_specs=[
            pl.BlockSpec((BB, s, h), lambda i: (i, 0, 0)),
            pl.BlockSpec((BB, s, 1), lambda i: (i, 0, 0)),
            pl.BlockSpec((1, s, h), lambda i: (0, 0, 0)),
            pl.BlockSpec((1, 1, h), lambda i: (0, 0, 0)),
            pl.BlockSpec((1, 1, h), lambda i: (0, 0, 0)),
            pl.BlockSpec((1, 1, h), lambda i: (0, 0, 0)),
        ],
        out_specs=pl.BlockSpec((BB, s, h), lambda i: (i, 0, 0)),
        out_shape=jax.ShapeDtypeStruct((b, s, h), jnp.float32),
        interpret=interpret,
    )(gathered, tt_f, base, diff, ln_w, ln_b)


def kernel(input_ids, token_type_ids, W_word, W_pos, W_type, ln_w, ln_b):
    b, s = input_ids.shape
    flat_ids = input_ids.reshape(-1).astype(jnp.int32)
    gathered = _sc_gather(W_word, flat_ids).reshape(b, s, HIDDEN)
    tt_f = token_type_ids.astype(jnp.float32)[:, :, None]
    base = (W_pos[:s] + W_type[0][None, :])[None, :, :]
    diff = (W_type[1] - W_type[0])[None, None, :]
    return _tc_addln(gathered, tt_f, base, diff,
                     ln_w.reshape(1, 1, HIDDEN), ln_b.reshape(1, 1, HIDDEN))
